# Initial kernel scaffold; baseline (speedup 1.0000x reference)
import jax
import jax.numpy as jnp
from jax.experimental import pallas as pl


def kernel(x, edge_index, seq_batch_node_id, c0_W1, c0_b1, c0_g1, c0_bt1, c0_W2, c0_b2, c0_g2, c0_bt2, c1_W1, c1_b1, c1_g1, c1_bt1, c1_W2, c1_b2, c1_g2, c1_bt2):
    raise NotImplementedError("write your pallas kernel here")


# speedup vs baseline: 5.9042x; 5.9042x over previous
import functools

import jax
import jax.numpy as jnp
from jax import lax
from jax.experimental import pallas as pl
from jax.experimental.pallas import tpu as pltpu
from jax.experimental.pallas import tpu_sc as plsc

N = 10000
E = 320000
D = 128
H = 128
G = 64
BN_EPS = 1e-5

NC = 2
NS = 16
CHUNK = 128
NCHUNKS = E // CHUNK
BASE = NCHUNKS // (NC * NS)
REM = NCHUNKS % (NC * NS)
ROWS_PER_TILE = 624
TAIL_OFF = ROWS_PER_TILE * NS
TAIL = N - TAIL_OFF


def _sc_edge_agg(src, dst, h, zeros):
    mesh = plsc.VectorSubcoreMesh(core_axis_name="c", subcore_axis_name="s")

    @functools.partial(
        pl.kernel,
        out_type=jax.ShapeDtypeStruct((NC, N, D), jnp.float32),
        mesh=mesh,
        scratch_types=[
            pltpu.VMEM((CHUNK,), jnp.int32),
            pltpu.VMEM((CHUNK,), jnp.int32),
            pltpu.VMEM((CHUNK, D), jnp.float32),
            pltpu.VMEM_SHARED((N, D), jnp.float32),
            pltpu.SemaphoreType.DMA,
        ],
    )
    def k(src_hbm, dst_hbm, h_hbm, zeros_hbm, out_hbm, sidx, didx, rows, agg_sh, sem):
        cid = lax.axis_index("c")
        sid = lax.axis_index("s")
        wid = sid * NC + cid
        r0 = sid * ROWS_PER_TILE
        pltpu.sync_copy(zeros_hbm.at[pl.ds(r0, ROWS_PER_TILE)],
                        agg_sh.at[pl.ds(r0, ROWS_PER_TILE)])

        @pl.when(sid == NS - 1)
        def _():
            pltpu.sync_copy(zeros_hbm.at[pl.ds(TAIL_OFF, TAIL)],
                            agg_sh.at[pl.ds(TAIL_OFF, TAIL)])

        plsc.subcore_barrier()

        nchunks = jnp.where(wid < REM, BASE + 1, BASE)
        start = wid * BASE + jnp.minimum(wid, REM)

        def body(i, carry):
            off = pl.multiple_of((start + i) * CHUNK, CHUNK)
            pltpu.sync_copy(src_hbm.at[pl.ds(off, CHUNK)], sidx)
            pltpu.sync_copy(dst_hbm.at[pl.ds(off, CHUNK)], didx)
            pltpu.async_copy(h_hbm.at[sidx], rows, sem).wait()
            pltpu.sync_copy(rows, agg_sh.at[didx], add=True)
            return carry

        lax.fori_loop(0, nchunks, body, 0)
        plsc.subcore_barrier()
        pltpu.sync_copy(agg_sh.at[pl.ds(r0, ROWS_PER_TILE)],
                        out_hbm.at[cid, pl.ds(r0, ROWS_PER_TILE)])

        @pl.when(sid == NS - 1)
        def _():
            pltpu.sync_copy(agg_sh.at[pl.ds(TAIL_OFF, TAIL)],
                            out_hbm.at[cid, pl.ds(TAIL_OFF, TAIL)])

    return k(src, dst, h, zeros)


def _mlp_pool_body(x_ref, agg_ref, seg_ref,
                   W1_ref, b1_ref, g1_ref, bt1_ref,
                   W2_ref, b2_ref, g2_ref, bt2_ref,
                   h_out_ref, pool_ref):
    h = x_ref[...] + agg_ref[0] + agg_ref[1]
    y = jnp.dot(h, W1_ref[...], preferred_element_type=jnp.float32) + b1_ref[...]
    mean = jnp.mean(y, axis=0, keepdims=True)
    var = jnp.mean((y - mean) * (y - mean), axis=0, keepdims=True)
    y = g1_ref[...] * (y - mean) * lax.rsqrt(var + BN_EPS) + bt1_ref[...]
    y = jnp.maximum(y, 0.0)
    z = jnp.dot(y, W2_ref[...], preferred_element_type=jnp.float32) + b2_ref[...]
    mean = jnp.mean(z, axis=0, keepdims=True)
    var = jnp.mean((z - mean) * (z - mean), axis=0, keepdims=True)
    z = g2_ref[...] * (z - mean) * lax.rsqrt(var + BN_EPS) + bt2_ref[...]
    z = jnp.maximum(z, 0.0)
    h_out_ref[...] = z
    gids = lax.broadcasted_iota(jnp.int32, (G, N), 0)
    onehot = (gids == seg_ref[...]).astype(jnp.float32)
    pool_ref[...] = jnp.dot(onehot, z, preferred_element_type=jnp.float32)


def _mlp_pool(h, agg, seg, W1, b1, g1, bt1, W2, b2, g2, bt2):
    return pl.pallas_call(
        _mlp_pool_body,
        out_shape=[
            jax.ShapeDtypeStruct((N, H), jnp.float32),
            jax.ShapeDtypeStruct((G, H), jnp.float32),
        ],
    )(h, agg, seg,
      W1, b1.reshape(1, H), g1.reshape(1, H), bt1.reshape(1, H),
      W2, b2.reshape(1, H), g2.reshape(1, H), bt2.reshape(1, H))


def kernel(x, edge_index, seq_batch_node_id,
           c0_W1, c0_b1, c0_g1, c0_bt1, c0_W2, c0_b2, c0_g2, c0_bt2,
           c1_W1, c1_b1, c1_g1, c1_bt1, c1_W2, c1_b2, c1_g2, c1_bt2):
    src = edge_index[0]
    dst = edge_index[1]
    seg = seq_batch_node_id.reshape(1, N)
    zeros = jnp.zeros((N, D), jnp.float32)

    agg0 = _sc_edge_agg(src, dst, x, zeros)
    h0, p0 = _mlp_pool(x, agg0, seg,
                       c0_W1, c0_b1, c0_g1, c0_bt1, c0_W2, c0_b2, c0_g2, c0_bt2)
    agg1 = _sc_edge_agg(src, dst, h0, zeros)
    _, p1 = _mlp_pool(h0, agg1, seg,
                      c1_W1, c1_b1, c1_g1, c1_bt1, c1_W2, c1_b2, c1_g2, c1_bt2)
    return jnp.concatenate([p0, p1], axis=1)
